# baseline (speedup 1.0000x reference)
import functools

import jax
import jax.numpy as jnp
from jax import lax
from jax.experimental import pallas as pl
from jax.experimental.pallas import tpu as pltpu
from jax.experimental.pallas import tpu_sc as plsc

_K = 64
_N = 50000
_D = 256
_T = 80
_NCHUNK = _N // _T
_NW = 32
_BASE_TRIPS = _NCHUNK // _NW
_EXTRA = _NCHUNK % _NW

_mesh = plsc.VectorSubcoreMesh(core_axis_name="c", subcore_axis_name="s")


@functools.partial(
    pl.kernel,
    out_type=jax.ShapeDtypeStruct((_NW, 16), jnp.float32),
    mesh=_mesh,
    scratch_types=[
        pltpu.VMEM((_T, _D), jnp.float32),
        pltpu.VMEM((_T,), jnp.int32),
        pltpu.VMEM((_K, _D), jnp.float32),
        pltpu.VMEM((16,), jnp.float32),
    ],
    compiler_params=pltpu.CompilerParams(
        use_tc_tiling_on_sc=False, needs_layout_passes=False),
)
def _sc_loss(x_hbm, a_hbm, c_hbm, out_hbm, x_v, a_v, c_v, p_v):
    wid = lax.axis_index("s") * 2 + lax.axis_index("c")
    pltpu.sync_copy(c_hbm, c_v)

    lanes = lax.broadcasted_iota(jnp.int32, (16,), 0)
    n_my = jnp.where(wid < _EXTRA, _BASE_TRIPS + 1, _BASE_TRIPS)

    def chunk_body(i, acc):
        off = (wid + i * _NW) * _T
        pltpu.sync_copy(x_hbm.at[pl.ds(off, _T), :], x_v)
        pltpu.sync_copy(a_hbm.at[pl.ds(off, _T)], a_v)
        for g in range(_T // 16):
            pvec = lanes + (g * 16)
            va = a_v[pl.ds(g * 16, 16)]

            def dim_body(d, acc_in):
                dsplat = jnp.full((16,), d, jnp.int32)
                vx = plsc.load_gather(x_v, [pvec, dsplat])
                vc = plsc.load_gather(c_v, [va, dsplat])
                diff = vx - vc
                return acc_in + diff * diff

            acc = lax.fori_loop(0, _D, dim_body, acc)
        return acc

    acc = lax.fori_loop(0, n_my, chunk_body, jnp.zeros((16,), jnp.float32))
    p_v[...] = acc
    pltpu.sync_copy(p_v, out_hbm.at[wid])


def kernel(x, cluster_assignments, cluster_centers):
    partials = _sc_loss(x, cluster_assignments, cluster_centers)
    return jnp.sum(partials)


# speedup vs baseline: 1.0002x; 1.0002x over previous
import functools

import jax
import jax.numpy as jnp
from jax import lax
from jax.experimental import pallas as pl
from jax.experimental.pallas import tpu as pltpu
from jax.experimental.pallas import tpu_sc as plsc

_K = 64
_N = 50000
_D = 256
_T = 80
_NCHUNK = _N // _T
_NW = 32
_BASE_TRIPS = _NCHUNK // _NW
_EXTRA = _NCHUNK % _NW
_UNROLL = 32

_mesh = plsc.VectorSubcoreMesh(core_axis_name="c", subcore_axis_name="s")


@functools.partial(
    pl.kernel,
    out_type=jax.ShapeDtypeStruct((_NW, 16), jnp.float32),
    mesh=_mesh,
    scratch_types=[
        pltpu.VMEM((_T * _D,), jnp.float32),
        pltpu.VMEM((_T,), jnp.int32),
        pltpu.VMEM((_K * _D,), jnp.float32),
        pltpu.VMEM((16,), jnp.float32),
    ],
    compiler_params=pltpu.CompilerParams(
        use_tc_tiling_on_sc=False, needs_layout_passes=False),
)
def _sc_loss(x_hbm, a_hbm, c_hbm, out_hbm, x_v, a_v, c_v, p_v):
    wid = lax.axis_index("s") * 2 + lax.axis_index("c")
    pltpu.sync_copy(c_hbm, c_v)

    lanes = lax.broadcasted_iota(jnp.int32, (16,), 0)
    ones = jnp.ones((16,), jnp.int32)
    n_my = jnp.where(wid < _EXTRA, _BASE_TRIPS + 1, _BASE_TRIPS)

    def chunk_body(i, acc):
        off = (wid + i * _NW) * _T
        pltpu.sync_copy(x_hbm.at[pl.ds(off * _D, _T * _D)], x_v)
        pltpu.sync_copy(a_hbm.at[pl.ds(off, _T)], a_v)
        for g in range(_T // 16):
            xi0 = lanes * _D + (g * 16 * _D)
            ci0 = a_v[pl.ds(g * 16, 16)] * _D

            def dim_blk(b, carry):
                xi, ci, acc_in = carry
                for _ in range(_UNROLL):
                    vx = plsc.load_gather(x_v, [xi])
                    vc = plsc.load_gather(c_v, [ci])
                    diff = vx - vc
                    acc_in = acc_in + diff * diff
                    xi = xi + ones
                    ci = ci + ones
                return xi, ci, acc_in

            _, _, acc = lax.fori_loop(0, _D // _UNROLL, dim_blk,
                                      (xi0, ci0, acc))
        return acc

    acc = lax.fori_loop(0, n_my, chunk_body, jnp.zeros((16,), jnp.float32))
    p_v[...] = acc
    pltpu.sync_copy(p_v, out_hbm.at[wid])


def kernel(x, cluster_assignments, cluster_centers):
    partials = _sc_loss(x.reshape(-1), cluster_assignments,
                        cluster_centers.reshape(-1))
    return jnp.sum(partials)


# speedup vs baseline: 2.1768x; 2.1763x over previous
import functools

import jax
import jax.numpy as jnp
from jax import lax
from jax.experimental import pallas as pl
from jax.experimental.pallas import tpu as pltpu
from jax.experimental.pallas import tpu_sc as plsc

_K = 64
_N = 50000
_D = 256
_DP = _D + 1
_T = 80
_NCHUNK = _N // _T
_NW = 32
_BASE_TRIPS = _NCHUNK // _NW
_EXTRA = _NCHUNK % _NW
_UNROLL = 32

_mesh = plsc.VectorSubcoreMesh(core_axis_name="c", subcore_axis_name="s")


@functools.partial(
    pl.kernel,
    out_type=jax.ShapeDtypeStruct((_NW, 16), jnp.float32),
    mesh=_mesh,
    scratch_types=[
        pltpu.VMEM((_T, _DP), jnp.float32),
        pltpu.VMEM((_T,), jnp.int32),
        pltpu.VMEM((_K, _DP), jnp.float32),
        pltpu.VMEM((16,), jnp.float32),
    ],
    compiler_params=pltpu.CompilerParams(
        use_tc_tiling_on_sc=False, needs_layout_passes=False),
)
def _sc_loss(x_hbm, a_hbm, c_hbm, out_hbm, x_v, a_v, c_v, p_v):
    wid = lax.axis_index("s") * 2 + lax.axis_index("c")
    pltpu.sync_copy(c_hbm, c_v.at[:, pl.ds(0, _D)])

    lanes = lax.broadcasted_iota(jnp.int32, (16,), 0)
    ones = jnp.ones((16,), jnp.int32)
    n_my = jnp.where(wid < _EXTRA, _BASE_TRIPS + 1, _BASE_TRIPS)

    def chunk_body(i, acc):
        off = (wid + i * _NW) * _T
        pltpu.sync_copy(x_hbm.at[pl.ds(off, _T), :], x_v.at[:, pl.ds(0, _D)])
        pltpu.sync_copy(a_hbm.at[pl.ds(off, _T)], a_v)
        for g in range(_T // 16):
            pvec = lanes + (g * 16)
            va = a_v[pl.ds(g * 16, 16)]

            def dim_blk(b, carry):
                dvec, acc_in = carry
                for _ in range(_UNROLL):
                    vx = plsc.load_gather(x_v, [pvec, dvec])
                    vc = plsc.load_gather(c_v, [va, dvec])
                    diff = vx - vc
                    acc_in = acc_in + diff * diff
                    dvec = dvec + ones
                return dvec, acc_in

            _, acc = lax.fori_loop(0, _D // _UNROLL, dim_blk,
                                   (jnp.zeros((16,), jnp.int32), acc))
        return acc

    acc = lax.fori_loop(0, n_my, chunk_body, jnp.zeros((16,), jnp.float32))
    p_v[...] = acc
    pltpu.sync_copy(p_v, out_hbm.at[wid])


def kernel(x, cluster_assignments, cluster_centers):
    partials = _sc_loss(x, cluster_assignments, cluster_centers)
    return jnp.sum(partials)


# speedup vs baseline: 2.4652x; 1.1325x over previous
import functools

import jax
import jax.numpy as jnp
from jax import lax
from jax.experimental import pallas as pl
from jax.experimental.pallas import tpu as pltpu
from jax.experimental.pallas import tpu_sc as plsc

_K = 64
_N = 50000
_D = 256
_DP = _D + 1
_T = 80
_NCHUNK = _N // _T
_NW = 32
_SLOTS = -(-_NCHUNK // _NW)
_UNROLL = 32

_mesh = plsc.VectorSubcoreMesh(core_axis_name="c", subcore_axis_name="s")


@functools.partial(
    pl.kernel,
    out_type=jax.ShapeDtypeStruct((_NW, 16), jnp.float32),
    mesh=_mesh,
    scratch_types=[
        pltpu.VMEM((_T, _DP), jnp.float32),
        pltpu.VMEM((_T, _DP), jnp.float32),
        pltpu.VMEM((_T,), jnp.int32),
        pltpu.VMEM((_T,), jnp.int32),
        pltpu.VMEM((_K, _DP), jnp.float32),
        pltpu.VMEM((16,), jnp.float32),
        pltpu.SemaphoreType.DMA,
        pltpu.SemaphoreType.DMA,
        pltpu.SemaphoreType.DMA,
        pltpu.SemaphoreType.DMA,
    ],
    compiler_params=pltpu.CompilerParams(
        use_tc_tiling_on_sc=False, needs_layout_passes=False),
)
def _sc_loss(x_hbm, a_hbm, c_hbm, out_hbm,
             x_v0, x_v1, a_v0, a_v1, c_v, p_v,
             sx0, sx1, sa0, sa1):
    wid = lax.axis_index("s") * 2 + lax.axis_index("c")
    pltpu.sync_copy(c_hbm, c_v.at[:, pl.ds(0, _D)])

    lanes = lax.broadcasted_iota(jnp.int32, (16,), 0)
    ones = jnp.ones((16,), jnp.int32)
    xbufs = ((x_v0, a_v0, sx0, sa0), (x_v1, a_v1, sx1, sa1))

    def start(slot, buf):
        x_v, a_v, sx, sa = buf
        off = jnp.minimum(wid + slot * _NW, _NCHUNK - 1) * _T
        pltpu.make_async_copy(
            x_hbm.at[pl.ds(off, _T), :], x_v.at[:, pl.ds(0, _D)], sx).start()
        pltpu.make_async_copy(a_hbm.at[pl.ds(off, _T)], a_v, sa).start()

    def process(slot, buf, acc):
        x_v, a_v, sx, sa = buf
        off = jnp.minimum(wid + slot * _NW, _NCHUNK - 1) * _T
        pltpu.make_async_copy(
            x_hbm.at[pl.ds(off, _T), :], x_v.at[:, pl.ds(0, _D)], sx).wait()
        pltpu.make_async_copy(a_hbm.at[pl.ds(off, _T)], a_v, sa).wait()
        part = jnp.zeros((16,), jnp.float32)
        for g in range(_T // 16):
            pvec = lanes + (g * 16)
            va = a_v[pl.ds(g * 16, 16)]

            def dim_blk(b, carry):
                dvec, p_in = carry
                for _ in range(_UNROLL):
                    vx = plsc.load_gather(x_v, [pvec, dvec])
                    vc = plsc.load_gather(c_v, [va, dvec])
                    diff = vx - vc
                    p_in = p_in + diff * diff
                    dvec = dvec + ones
                return dvec, p_in

            _, part = lax.fori_loop(0, _D // _UNROLL, dim_blk,
                                    (jnp.zeros((16,), jnp.int32), part))
        valid = (wid + slot * _NW) < _NCHUNK
        return acc + jnp.where(valid, part, jnp.zeros((16,), jnp.float32))

    start(0, xbufs[0])

    def slot_pair(t, acc):
        s0 = t * 2
        start(s0 + 1, xbufs[1])
        acc = process(s0, xbufs[0], acc)
        start(s0 + 2, xbufs[0])
        acc = process(s0 + 1, xbufs[1], acc)
        return acc

    acc = lax.fori_loop(0, _SLOTS // 2, slot_pair,
                        jnp.zeros((16,), jnp.float32))
    pltpu.make_async_copy(
        x_hbm.at[pl.ds(0, _T), :], x_v0.at[:, pl.ds(0, _D)], sx0).wait()
    pltpu.make_async_copy(a_hbm.at[pl.ds(0, _T)], a_v0, sa0).wait()

    p_v[...] = acc
    pltpu.sync_copy(p_v, out_hbm.at[wid])


def kernel(x, cluster_assignments, cluster_centers):
    partials = _sc_loss(x, cluster_assignments, cluster_centers)
    return jnp.sum(partials)


# speedup vs baseline: 4.7692x; 1.9346x over previous
import functools

import jax
import jax.numpy as jnp
from jax import lax
from jax.experimental import pallas as pl
from jax.experimental.pallas import tpu as pltpu
from jax.experimental.pallas import tpu_sc as plsc

_K = 64
_N = 50000
_D = 256
_DP = _D + 1
_T = 80
_NW = 32
_SC_SLOTS = 2
_N_SC = _NW * _SC_SLOTS * _T
_N_TC = _N - _N_SC
_B_TC = 8976
_NB_TC = _N_TC // _B_TC
_UNROLL = 32

_mesh = plsc.VectorSubcoreMesh(core_axis_name="c", subcore_axis_name="s")


@functools.partial(
    pl.kernel,
    out_type=jax.ShapeDtypeStruct((_NW, 16), jnp.float32),
    mesh=_mesh,
    scratch_types=[
        pltpu.VMEM((_T, _DP), jnp.float32),
        pltpu.VMEM((_T, _DP), jnp.float32),
        pltpu.VMEM((_T,), jnp.int32),
        pltpu.VMEM((_T,), jnp.int32),
        pltpu.VMEM((_K, _DP), jnp.float32),
        pltpu.VMEM((16,), jnp.float32),
        pltpu.SemaphoreType.DMA,
        pltpu.SemaphoreType.DMA,
        pltpu.SemaphoreType.DMA,
        pltpu.SemaphoreType.DMA,
    ],
    compiler_params=pltpu.CompilerParams(
        use_tc_tiling_on_sc=False, needs_layout_passes=False),
)
def _sc_loss(x_hbm, a_hbm, c_hbm, out_hbm,
             x_v0, x_v1, a_v0, a_v1, c_v, p_v,
             sx0, sx1, sa0, sa1):
    wid = lax.axis_index("s") * 2 + lax.axis_index("c")
    pltpu.sync_copy(c_hbm, c_v.at[:, pl.ds(0, _D)])

    lanes = lax.broadcasted_iota(jnp.int32, (16,), 0)
    ones = jnp.ones((16,), jnp.int32)
    bufs = ((x_v0, a_v0, sx0, sa0), (x_v1, a_v1, sx1, sa1))

    def start(slot, buf):
        x_v, a_v, sx, sa = buf
        off = _N_TC + (wid * _SC_SLOTS + slot) * _T
        pltpu.make_async_copy(
            x_hbm.at[pl.ds(off, _T), :], x_v.at[:, pl.ds(0, _D)], sx).start()
        pltpu.make_async_copy(a_hbm.at[pl.ds(off, _T)], a_v, sa).start()

    def process(slot, buf, acc):
        x_v, a_v, sx, sa = buf
        off = _N_TC + (wid * _SC_SLOTS + slot) * _T
        pltpu.make_async_copy(
            x_hbm.at[pl.ds(off, _T), :], x_v.at[:, pl.ds(0, _D)], sx).wait()
        pltpu.make_async_copy(a_hbm.at[pl.ds(off, _T)], a_v, sa).wait()
        for g in range(_T // 16):
            pvec = lanes + (g * 16)
            va = a_v[pl.ds(g * 16, 16)]

            def dim_blk(b, carry):
                dvec, p_in = carry
                for _ in range(_UNROLL):
                    vx = plsc.load_gather(x_v, [pvec, dvec])
                    vc = plsc.load_gather(c_v, [va, dvec])
                    diff = vx - vc
                    p_in = p_in + diff * diff
                    dvec = dvec + ones
                return dvec, p_in

            _, acc = lax.fori_loop(0, _D // _UNROLL, dim_blk,
                                   (jnp.zeros((16,), jnp.int32), acc))
        return acc

    start(0, bufs[0])
    start(1, bufs[1])
    acc = process(0, bufs[0], jnp.zeros((16,), jnp.float32))
    acc = process(1, bufs[1], acc)

    p_v[...] = acc
    pltpu.sync_copy(p_v, out_hbm.at[wid])


def _tc_loss_block(x_ref, a_ref, c_ref, out_ref):
    i = pl.program_id(0)
    x = x_ref[...]
    a = a_ref[0]
    c = c_ref[...]

    xs = jnp.sum(x * x)
    scores = jax.lax.dot_general(
        c, x, (((1,), (1,)), ((), ())), preferred_element_type=jnp.float32)
    cn = jnp.sum(c * c, axis=1, keepdims=True)
    m = cn - 2.0 * scores
    row = jax.lax.broadcasted_iota(jnp.int32, (_K, _B_TC), 0)
    oh = row == a
    s = jax.lax.broadcast(xs + jnp.sum(jnp.where(oh, m, 0.0)), (1, 1))

    @pl.when(i == 0)
    def _():
        out_ref[...] = s

    @pl.when(i != 0)
    def _():
        out_ref[...] += s


def _tc_loss(x, a3, c):
    return pl.pallas_call(
        _tc_loss_block,
        grid=(_NB_TC,),
        in_specs=[
            pl.BlockSpec((_B_TC, _D), lambda i: (i, 0)),
            pl.BlockSpec((1, 1, _B_TC), lambda i: (i, 0, 0)),
            pl.BlockSpec((_K, _D), lambda i: (0, 0)),
        ],
        out_specs=pl.BlockSpec((1, 1), lambda i: (0, 0)),
        out_shape=jax.ShapeDtypeStruct((1, 1), jnp.float32),
    )(x, a3, c)


def kernel(x, cluster_assignments, cluster_centers):
    a3 = cluster_assignments[:_N_TC].reshape(_NB_TC, 1, _B_TC)
    sc_partials = _sc_loss(x, cluster_assignments, cluster_centers)
    tc_part = _tc_loss(x, a3, cluster_centers)
    return tc_part[0, 0] + jnp.sum(sc_partials)
